# baseline (speedup 1.0000x reference)
import jax
import jax.numpy as jnp
import numpy as np
from jax.experimental import pallas as pl
from jax.experimental.pallas import tpu as pltpu

_B, _F = 65536, 300
_CR = 512
_D = 4
_NC = _B // _CR


def _chunk_fn(x, w, fr):
    b = jnp.dot(x, w, preferred_element_type=jnp.float32)
    r = x.shape[0]
    neg = jnp.full((r, 1), -jnp.inf, dtype=jnp.float32)
    bl = jnp.concatenate([neg, b[:, :-1]], axis=1)
    br = jnp.concatenate([b[:, 1:], neg], axis=1)
    is_peak = (b > bl) & (b > br)

    iota = jax.lax.broadcasted_iota(jnp.int32, (1, _F), 1).astype(jnp.float32)
    nf = jnp.float32(_F)
    idx = jnp.where(is_peak, iota, nf)
    p1 = jnp.min(idx, axis=1, keepdims=True)
    idx2 = jnp.where(iota > p1, idx, nf)
    p2 = jnp.min(idx2, axis=1, keepdims=True)
    end = jnp.where(p2 < nf, jnp.floor((p1 + p2) * 0.5), nf - 1.0)

    logits = jnp.where(iota < end, b, -jnp.inf)
    m = jnp.max(logits, axis=1, keepdims=True)
    e = jnp.exp(logits - m)
    s = jnp.sum(e, axis=1, keepdims=True)
    sf = jnp.sum(e * fr, axis=1, keepdims=True)
    return -(sf / s)


def _body(x_hbm, w_ref, fr_ref, o_ref, buf, sems):
    c = pl.program_id(0)

    @pl.when(c == 0)
    def _():
        for k in range(_D - 1):
            pltpu.make_async_copy(
                x_hbm.at[pl.ds(k * _CR, _CR), :], buf.at[k], sems.at[k]
            ).start()

    nxt = c + _D - 1

    @pl.when(nxt < _NC)
    def _():
        slot_n = jax.lax.rem(nxt, _D)
        pltpu.make_async_copy(
            x_hbm.at[pl.ds(nxt * _CR, _CR), :], buf.at[slot_n], sems.at[slot_n]
        ).start()

    slot = jax.lax.rem(c, _D)
    pltpu.make_async_copy(buf.at[slot], buf.at[slot], sems.at[slot]).wait()
    o_ref[...] = _chunk_fn(buf[slot], w_ref[...], fr_ref[...])


def kernel(fr_funcs, freqs, kernel):
    ii = jnp.arange(_F, dtype=jnp.int32)[:, None]
    jj = jnp.arange(_F, dtype=jnp.int32)[None, :]
    t = ii - jj + 3
    w = jnp.zeros((_F, _F), dtype=jnp.float32)
    for tap in range(7):
        w = w + jnp.where(t == tap, kernel[tap].astype(jnp.float32), 0.0)

    freqs2 = freqs.astype(jnp.float32).reshape(1, _F)

    out = pl.pallas_call(
        _body,
        grid=(_NC,),
        in_specs=[
            pl.BlockSpec(memory_space=pl.ANY),
            pl.BlockSpec((_F, _F), lambda i: (0, 0)),
            pl.BlockSpec((1, _F), lambda i: (0, 0)),
        ],
        out_specs=pl.BlockSpec((_CR, 1), lambda i: (i, 0)),
        out_shape=jax.ShapeDtypeStruct((_B, 1), jnp.float32),
        scratch_shapes=[
            pltpu.VMEM((_D, _CR, _F), jnp.float32),
            pltpu.SemaphoreType.DMA((_D,)),
        ],
        compiler_params=pltpu.CompilerParams(
            dimension_semantics=("arbitrary",),
        ),
    )(fr_funcs, w, freqs2)
    return out[:, 0]


# speedup vs baseline: 1.0952x; 1.0952x over previous
import jax
import jax.numpy as jnp
import numpy as np
from jax.experimental import pallas as pl
from jax.experimental.pallas import tpu as pltpu

_B, _F = 65536, 300
_CR = 2048
_D = 3
_NC = _B // _CR


def _chunk_fn(x, w, fr):
    b = jnp.dot(x, w, preferred_element_type=jnp.float32)
    r = x.shape[0]
    neg = jnp.full((r, 1), -jnp.inf, dtype=jnp.float32)
    bl = jnp.concatenate([neg, b[:, :-1]], axis=1)
    br = jnp.concatenate([b[:, 1:], neg], axis=1)
    is_peak = (b > bl) & (b > br)

    iota = jax.lax.broadcasted_iota(jnp.int32, (1, _F), 1).astype(jnp.float32)
    nf = jnp.float32(_F)
    idx = jnp.where(is_peak, iota, nf)
    p1 = jnp.min(idx, axis=1, keepdims=True)
    idx2 = jnp.where(iota > p1, idx, nf)
    p2 = jnp.min(idx2, axis=1, keepdims=True)
    end = jnp.where(p2 < nf, jnp.floor((p1 + p2) * 0.5), nf - 1.0)

    logits = jnp.where(iota < end, b, -jnp.inf)
    m = jnp.max(logits, axis=1, keepdims=True)
    e = jnp.exp(logits - m)
    s = jnp.sum(e, axis=1, keepdims=True)
    sf = jnp.sum(e * fr, axis=1, keepdims=True)
    return -(sf / s)


def _body(x_hbm, w_ref, fr_ref, o_ref, buf, sems):
    c = pl.program_id(0)

    @pl.when(c == 0)
    def _():
        for k in range(_D - 1):
            pltpu.make_async_copy(
                x_hbm.at[pl.ds(k * _CR, _CR), :], buf.at[k], sems.at[k]
            ).start()

    nxt = c + _D - 1

    @pl.when(nxt < _NC)
    def _():
        slot_n = jax.lax.rem(nxt, _D)
        pltpu.make_async_copy(
            x_hbm.at[pl.ds(nxt * _CR, _CR), :], buf.at[slot_n], sems.at[slot_n]
        ).start()

    slot = jax.lax.rem(c, _D)
    pltpu.make_async_copy(buf.at[slot], buf.at[slot], sems.at[slot]).wait()
    o_ref[...] = _chunk_fn(buf[slot], w_ref[...], fr_ref[...])


def kernel(fr_funcs, freqs, kernel):
    ii = jnp.arange(_F, dtype=jnp.int32)[:, None]
    jj = jnp.arange(_F, dtype=jnp.int32)[None, :]
    t = ii - jj + 3
    w = jnp.zeros((_F, _F), dtype=jnp.float32)
    for tap in range(7):
        w = w + jnp.where(t == tap, kernel[tap].astype(jnp.float32), 0.0)

    freqs2 = freqs.astype(jnp.float32).reshape(1, _F)

    out = pl.pallas_call(
        _body,
        grid=(_NC,),
        in_specs=[
            pl.BlockSpec(memory_space=pl.ANY),
            pl.BlockSpec((_F, _F), lambda i: (0, 0)),
            pl.BlockSpec((1, _F), lambda i: (0, 0)),
        ],
        out_specs=pl.BlockSpec((_CR, 1), lambda i: (i, 0)),
        out_shape=jax.ShapeDtypeStruct((_B, 1), jnp.float32),
        scratch_shapes=[
            pltpu.VMEM((_D, _CR, _F), jnp.float32),
            pltpu.SemaphoreType.DMA((_D,)),
        ],
        compiler_params=pltpu.CompilerParams(
            dimension_semantics=("arbitrary",),
        ),
    )(fr_funcs, w, freqs2)
    return out[:, 0]


# speedup vs baseline: 1.1577x; 1.0571x over previous
import jax
import jax.numpy as jnp
import numpy as np
from jax.experimental import pallas as pl
from jax.experimental.pallas import tpu as pltpu

_B, _F = 65536, 300
_ROWS = 2048


def _body(x_ref, w_ref, fr_ref, o_ref):
    x = x_ref[...]
    w = w_ref[...]
    b = jax.lax.dot_general(x, w, (((1,), (0,)), ((), ())), preferred_element_type=jnp.float32)

    r = x.shape[0]
    neg = jnp.full((r, 1), -jnp.inf, dtype=jnp.float32)
    bl = jnp.concatenate([neg, b[:, :-1]], axis=1)
    br = jnp.concatenate([b[:, 1:], neg], axis=1)
    is_peak = (b > bl) & (b > br)

    iota = jax.lax.broadcasted_iota(jnp.int32, (1, _F), 1).astype(jnp.float32)
    nf = jnp.float32(_F)
    idx = jnp.where(is_peak, iota, nf)
    p1 = jnp.min(idx, axis=1, keepdims=True)
    idx2 = jnp.where(iota > p1, idx, nf)
    p2 = jnp.min(idx2, axis=1, keepdims=True)
    end = jnp.where(p2 < nf, jnp.floor((p1 + p2) * 0.5), nf - 1.0)

    e = jnp.where(iota < end, jnp.exp(b), 0.0)
    s = jnp.sum(e, axis=1, keepdims=True)
    sf = jnp.sum(e * fr_ref[...], axis=1, keepdims=True)
    o_ref[...] = -(sf / s)


def kernel(fr_funcs, freqs, kernel):
    ii = jnp.arange(_F, dtype=jnp.int32)[:, None]
    jj = jnp.arange(_F, dtype=jnp.int32)[None, :]
    t = ii - jj + 3
    w = jnp.zeros((_F, _F), dtype=jnp.float32)
    for tap in range(7):
        w = w + jnp.where(t == tap, kernel[tap].astype(jnp.float32), 0.0)

    freqs2 = freqs.astype(jnp.float32).reshape(1, _F)

    out = pl.pallas_call(
        _body,
        grid=(_B // _ROWS,),
        in_specs=[
            pl.BlockSpec((_ROWS, _F), lambda i: (i, 0)),
            pl.BlockSpec((_F, _F), lambda i: (0, 0)),
            pl.BlockSpec((1, _F), lambda i: (0, 0)),
        ],
        out_specs=pl.BlockSpec((_ROWS, 1), lambda i: (i, 0)),
        out_shape=jax.ShapeDtypeStruct((_B, 1), jnp.float32),
        compiler_params=pltpu.CompilerParams(
            dimension_semantics=("parallel",),
        ),
    )(fr_funcs, w, freqs2)
    return out[:, 0]
